# baseline (speedup 1.0000x reference)
import jax
import jax.numpy as jnp
from jax.experimental import pallas as pl

_BLK = 512


def _nc_body(x_ref, ct_ref, o_ref):
    xb = x_ref[...]
    ct = ct_ref[...]
    c_sq = jnp.sum(ct * ct, axis=0)
    mm = jax.lax.dot_general(
        xb, ct, (((1,), (0,)), ((), ())),
        precision=jax.lax.Precision.DEFAULT,
        preferred_element_type=jnp.float32)
    scores = c_sq[None, :] - 2.0 * mm
    idx = jnp.argmin(scores, axis=1)
    o_ref[...] = idx.astype(jnp.float32).reshape(1, 1, _BLK)


def kernel(x, clusters):
    bs, length, dim = x.shape
    k = clusters.shape[0]
    n = bs * length
    xf = x.reshape(n, dim)
    ct = clusters.T
    grid = n // _BLK
    out = pl.pallas_call(
        _nc_body,
        grid=(grid,),
        in_specs=[
            pl.BlockSpec((_BLK, dim), lambda i: (i, 0)),
            pl.BlockSpec((dim, k), lambda i: (0, 0)),
        ],
        out_specs=pl.BlockSpec((1, 1, _BLK), lambda i: (i, 0, 0)),
        out_shape=jax.ShapeDtypeStruct((grid, 1, _BLK), jnp.float32),
    )(xf, ct)
    return out.reshape(bs, length)


# speedup vs baseline: 1.2861x; 1.2861x over previous
import jax
import jax.numpy as jnp
from jax.experimental import pallas as pl

_BLK = 1024


def _nc_body(x_ref, ct2_ref, o_ref):
    xb = x_ref[...]
    ct2 = ct2_ref[...]
    k = ct2.shape[1]
    c_sq = jnp.sum(ct2 * ct2, axis=0) * 0.25
    mm2 = jax.lax.dot_general(
        xb, ct2, (((1,), (0,)), ((), ())),
        preferred_element_type=jnp.float32)
    scores = mm2 + c_sq[None, :]
    row_min = jnp.min(scores, axis=1, keepdims=True)
    eqf = jnp.where(scores == row_min, 1.0, 0.0)
    ki = jax.lax.broadcasted_iota(jnp.int32, (k, 2), 0)
    sel = jax.lax.broadcasted_iota(jnp.int32, (k, 2), 1)
    iota2 = jnp.where(sel == 0, ki - (ki & 1), ki & 1).astype(jnp.float32)
    parts = jax.lax.dot_general(
        eqf, iota2, (((1,), (0,)), ((), ())),
        preferred_element_type=jnp.float32)
    o_ref[...] = parts[:, 0:1] + parts[:, 1:2]


def kernel(x, clusters):
    bs, length, dim = x.shape
    k = clusters.shape[0]
    n = bs * length
    xf = x.reshape(n, dim)
    ct2 = -2.0 * clusters.T
    grid = n // _BLK
    out = pl.pallas_call(
        _nc_body,
        grid=(grid,),
        in_specs=[
            pl.BlockSpec((_BLK, dim), lambda i: (i, 0)),
            pl.BlockSpec((dim, k), lambda i: (0, 0)),
        ],
        out_specs=pl.BlockSpec((_BLK, 1), lambda i: (i, 0)),
        out_shape=jax.ShapeDtypeStruct((n, 1), jnp.float32),
    )(xf, ct2)
    return out.reshape(bs, length)


# speedup vs baseline: 1.6916x; 1.3153x over previous
import jax
import jax.numpy as jnp
from jax.experimental import pallas as pl

_BLK = 4096


def _nc_body(x_ref, ct2_ref, o_ref):
    xb = x_ref[...]
    ct2 = ct2_ref[...]
    k = ct2.shape[1]
    c_sq = jnp.sum(ct2 * ct2, axis=0) * 0.25
    mm2 = jax.lax.dot_general(
        xb, ct2, (((1,), (0,)), ((), ())),
        preferred_element_type=jnp.float32)
    scores = mm2 + c_sq[None, :]
    row_min = jnp.min(scores, axis=1, keepdims=True)
    eqf = jnp.where(scores == row_min, 1.0, 0.0)
    ki = jax.lax.broadcasted_iota(jnp.int32, (k, 2), 0)
    sel = jax.lax.broadcasted_iota(jnp.int32, (k, 2), 1)
    iota2 = jnp.where(sel == 0, ki - (ki & 1), ki & 1).astype(jnp.float32)
    parts = jax.lax.dot_general(
        eqf, iota2, (((1,), (0,)), ((), ())),
        preferred_element_type=jnp.float32)
    o_ref[...] = parts[:, 0:1] + parts[:, 1:2]


def kernel(x, clusters):
    bs, length, dim = x.shape
    k = clusters.shape[0]
    n = bs * length
    xf = x.reshape(n, dim)
    ct2 = -2.0 * clusters.T
    grid = n // _BLK
    out = pl.pallas_call(
        _nc_body,
        grid=(grid,),
        in_specs=[
            pl.BlockSpec((_BLK, dim), lambda i: (i, 0)),
            pl.BlockSpec((dim, k), lambda i: (0, 0)),
        ],
        out_specs=pl.BlockSpec((_BLK, 1), lambda i: (i, 0)),
        out_shape=jax.ShapeDtypeStruct((n, 1), jnp.float32),
    )(xf, ct2)
    return out.reshape(bs, length)


# speedup vs baseline: 2.1789x; 1.2880x over previous
import jax
import jax.numpy as jnp
from jax.experimental import pallas as pl

_BLK = 4096


def _nc_body(x_ref, c_ref, o_ref):
    xb = x_ref[...]
    c2 = c_ref[...] * -2.0
    k = c2.shape[0]
    c_sq = jnp.sum(c2 * c2, axis=1) * 0.25
    mm2t = jax.lax.dot_general(
        c2, xb, (((1,), (1,)), ((), ())),
        preferred_element_type=jnp.float32)
    scores = mm2t + c_sq[:, None]
    col_min = jnp.min(scores, axis=0, keepdims=True)
    eqf = jnp.where(scores == col_min, 1.0, 0.0)
    ki = jax.lax.broadcasted_iota(jnp.int32, (2, k), 1)
    sel = jax.lax.broadcasted_iota(jnp.int32, (2, k), 0)
    iota2 = jnp.where(sel == 0, ki - (ki & 1), ki & 1).astype(jnp.float32)
    parts = jax.lax.dot_general(
        iota2, eqf, (((1,), (0,)), ((), ())),
        preferred_element_type=jnp.float32)
    o_ref[...] = (parts[0:1, :] + parts[1:2, :]).reshape(1, 1, _BLK)


def kernel(x, clusters):
    bs, length, dim = x.shape
    k = clusters.shape[0]
    n = bs * length
    xf = x.reshape(n, dim)
    grid = n // _BLK
    out = pl.pallas_call(
        _nc_body,
        grid=(grid,),
        in_specs=[
            pl.BlockSpec((_BLK, dim), lambda i: (i, 0)),
            pl.BlockSpec((k, dim), lambda i: (0, 0)),
        ],
        out_specs=pl.BlockSpec((1, 1, _BLK), lambda i: (i, 0, 0)),
        out_shape=jax.ShapeDtypeStruct((grid, 1, _BLK), jnp.float32),
    )(xf, clusters)
    return out.reshape(bs, length)


# speedup vs baseline: 3.6008x; 1.6525x over previous
import jax
import jax.numpy as jnp
from jax.experimental import pallas as pl
from jax.experimental.pallas import tpu as pltpu

_B_ROWS = 8


def _nc_body(x_ref, ct_ref, o_ref, c2_ref, csq_ref):
    k = ct_ref.shape[1]
    length = x_ref.shape[2]

    @pl.when(pl.program_id(0) == 0)
    def _init():
        c2t = jnp.transpose(ct_ref[...] * -2.0)
        c2_ref[...] = c2t
        csq_ref[...] = jnp.sum(c2t * c2t, axis=1, keepdims=True) * 0.25

    c2 = c2_ref[...]
    csq = csq_ref[...]
    ki = jax.lax.broadcasted_iota(jnp.int32, (2, k), 1)
    sel = jax.lax.broadcasted_iota(jnp.int32, (2, k), 0)
    iota2 = jnp.where(sel == 0, ki - (ki & 1), ki & 1).astype(jnp.float32)

    for r in range(_B_ROWS):
        xt = x_ref[r]
        mm2t = jax.lax.dot_general(
            c2, xt, (((1,), (0,)), ((), ())),
            preferred_element_type=jnp.float32)
        scores = mm2t + csq
        col_min = jnp.min(scores, axis=0, keepdims=True)
        eqf = jnp.where(scores == col_min, 1.0, 0.0)
        parts = jax.lax.dot_general(
            iota2, eqf, (((1,), (0,)), ((), ())),
            preferred_element_type=jnp.float32)
        o_ref[r, :] = (parts[0, :] + parts[1, :]).reshape(length)


def kernel(x, clusters):
    bs, length, dim = x.shape
    k = clusters.shape[0]
    xt = jnp.transpose(x, (0, 2, 1))
    ct = clusters.T
    grid = bs // _B_ROWS
    out = pl.pallas_call(
        _nc_body,
        grid=(grid,),
        in_specs=[
            pl.BlockSpec((_B_ROWS, dim, length), lambda i: (i, 0, 0)),
            pl.BlockSpec((dim, k), lambda i: (0, 0)),
        ],
        out_specs=pl.BlockSpec((_B_ROWS, length), lambda i: (i, 0)),
        out_shape=jax.ShapeDtypeStruct((bs, length), jnp.float32),
        scratch_shapes=[
            pltpu.VMEM((k, dim), jnp.float32),
            pltpu.VMEM((k, 1), jnp.float32),
        ],
    )(xt, ct)
    return out


# speedup vs baseline: 3.6169x; 1.0045x over previous
import jax
import jax.numpy as jnp
from jax.experimental import pallas as pl
from jax.experimental.pallas import tpu as pltpu

_B_ROWS = 16


def _nc_body(x_ref, ct_ref, o_ref, c2_ref, csq_ref):
    k = ct_ref.shape[1]
    length = x_ref.shape[2]

    @pl.when(pl.program_id(0) == 0)
    def _init():
        c2t = jnp.transpose(ct_ref[...] * -2.0)
        c2_ref[...] = c2t
        csq_ref[...] = jnp.sum(c2t * c2t, axis=1, keepdims=True) * 0.25

    c2 = c2_ref[...]
    csq = csq_ref[...]
    ki = jax.lax.broadcasted_iota(jnp.int32, (2, k), 1)
    sel = jax.lax.broadcasted_iota(jnp.int32, (2, k), 0)
    iota2 = jnp.where(sel == 0, ki - (ki & 1), ki & 1).astype(jnp.float32)

    for r in range(_B_ROWS):
        xt = x_ref[r]
        mm2t = jax.lax.dot_general(
            c2, xt, (((1,), (0,)), ((), ())),
            preferred_element_type=jnp.float32)
        scores = mm2t + csq
        col_min = jnp.min(scores, axis=0, keepdims=True)
        eqf = jnp.where(scores == col_min, 1.0, 0.0)
        parts = jax.lax.dot_general(
            iota2, eqf, (((1,), (0,)), ((), ())),
            preferred_element_type=jnp.float32)
        o_ref[r, :] = (parts[0, :] + parts[1, :]).reshape(length)


def kernel(x, clusters):
    bs, length, dim = x.shape
    k = clusters.shape[0]
    xt = jnp.transpose(x, (0, 2, 1))
    ct = clusters.T
    grid = bs // _B_ROWS
    out = pl.pallas_call(
        _nc_body,
        grid=(grid,),
        in_specs=[
            pl.BlockSpec((_B_ROWS, dim, length), lambda i: (i, 0, 0)),
            pl.BlockSpec((dim, k), lambda i: (0, 0)),
        ],
        out_specs=pl.BlockSpec((_B_ROWS, length), lambda i: (i, 0)),
        out_shape=jax.ShapeDtypeStruct((bs, length), jnp.float32),
        scratch_shapes=[
            pltpu.VMEM((k, dim), jnp.float32),
            pltpu.VMEM((k, 1), jnp.float32),
        ],
    )(xt, ct)
    return out


# speedup vs baseline: 6.1231x; 1.6929x over previous
import jax
import jax.numpy as jnp
from jax.experimental import pallas as pl
from jax.experimental.pallas import tpu as pltpu

_B_ROWS = 16


def _nc_body(x_ref, ct_ref, o_ref, c2_ref, csq_ref):
    k = ct_ref.shape[1]
    length = x_ref.shape[2]

    @pl.when(pl.program_id(0) == 0)
    def _init():
        c2t = jnp.transpose(ct_ref[...] * -2.0)
        c2_ref[...] = c2t
        csq_ref[...] = jnp.sum(c2t * c2t, axis=1, keepdims=True) * 0.25

    c2 = c2_ref[...]
    csq = csq_ref[...]

    for r in range(_B_ROWS):
        xt = x_ref[r]
        mm2t = jax.lax.dot_general(
            c2, xt, (((1,), (0,)), ((), ())),
            preferred_element_type=jnp.float32)
        scores = mm2t + csq
        o_ref[r, :] = jnp.argmin(scores, axis=0).astype(jnp.float32)


def kernel(x, clusters):
    bs, length, dim = x.shape
    k = clusters.shape[0]
    xt = jnp.transpose(x, (0, 2, 1))
    ct = clusters.T
    grid = bs // _B_ROWS
    out = pl.pallas_call(
        _nc_body,
        grid=(grid,),
        in_specs=[
            pl.BlockSpec((_B_ROWS, dim, length), lambda i: (i, 0, 0)),
            pl.BlockSpec((dim, k), lambda i: (0, 0)),
        ],
        out_specs=pl.BlockSpec((_B_ROWS, length), lambda i: (i, 0)),
        out_shape=jax.ShapeDtypeStruct((bs, length), jnp.float32),
        scratch_shapes=[
            pltpu.VMEM((k, dim), jnp.float32),
            pltpu.VMEM((k, 1), jnp.float32),
        ],
    )(xt, ct)
    return out
